# Initial kernel scaffold; baseline (speedup 1.0000x reference)
import jax
import jax.numpy as jnp
from jax.experimental import pallas as pl


def kernel(X, edge_index, edge_attr, W_heads, a_heads, W_out, a_out):
    raise NotImplementedError("write your pallas kernel here")


# speedup vs baseline: 5.6735x; 5.6735x over previous
import functools

import jax
import jax.numpy as jnp
from jax import lax
from jax.experimental import pallas as pl
from jax.experimental.pallas import tpu as pltpu
from jax.experimental.pallas import tpu_sc as plsc

N_NODES = 10000
NPAD = 10240
E_EDGES = 320000
EPAD = 327680
D_IN = 128
HID = 128
D_EDGE = 16
N_HEADS = 8
NS = 16
NC = 2
ACC_W = 144
CHUNK = 128
SUP = 4
EROWS = EPAD // CHUNK

_f32 = jnp.float32
_i32 = jnp.int32


def _elu(x):
    return jnp.where(x > 0, x, jnp.exp(jnp.minimum(x, 0.0)) - 1.0)


def _h_body(x_ref, w_ref, h_ref):
    h_ref[0] = jnp.dot(x_ref[...], w_ref[0], preferred_element_type=_f32)


def _tc_heads_matmul(Xp, W_heads):
    blk = 2048
    return pl.pallas_call(
        _h_body,
        grid=(N_HEADS, NPAD // blk),
        in_specs=[
            pl.BlockSpec((blk, D_IN), lambda i, j: (j, 0)),
            pl.BlockSpec((1, D_IN, HID), lambda i, j: (i, 0, 0)),
        ],
        out_specs=pl.BlockSpec((1, blk, HID), lambda i, j: (i, j, 0)),
        out_shape=jax.ShapeDtypeStruct((N_HEADS, NPAD, HID), _f32),
    )(Xp, W_heads)


def _s_body(x_ref, w_ref, a_ref, s_ref):
    W = w_ref[...]
    a = a_ref[...]
    a_l = a[:, :HID, 0]
    a_r = a[:, HID:2 * HID, 0]
    q_l = lax.dot_general(W, a_l, (((2,), (1,)), ((0,), (0,))),
                          preferred_element_type=_f32)
    q_r = lax.dot_general(W, a_r, (((2,), (1,)), ((0,), (0,))),
                          preferred_element_type=_f32)
    q = jnp.concatenate([q_l, q_r], axis=0)
    s_ref[...] = lax.dot_general(q, x_ref[...], (((1,), (1,)), ((), ())),
                                 preferred_element_type=_f32)


def _tc_node_scores(Xp, W_heads, a_heads):
    blk = 2048
    return pl.pallas_call(
        _s_body,
        grid=(NPAD // blk,),
        in_specs=[
            pl.BlockSpec((blk, D_IN), lambda j: (j, 0)),
            pl.BlockSpec((N_HEADS, D_IN, HID), lambda j: (0, 0, 0)),
            pl.BlockSpec((N_HEADS, 2 * HID + D_EDGE, 1), lambda j: (0, 0, 0)),
        ],
        out_specs=pl.BlockSpec((2 * N_HEADS, blk), lambda j: (0, j)),
        out_shape=jax.ShapeDtypeStruct((2 * N_HEADS, NPAD), _f32),
    )(Xp, W_heads, a_heads)


def _es_body(a_ref, e_ref, o_ref):
    o_ref[...] = lax.dot_general(a_ref[...], e_ref[...],
                                 (((1,), (1,)), ((), ())),
                                 preferred_element_type=_f32)


def _tc_edge_scores(A_all, eap):
    rows = A_all.shape[0]
    blk = 8192
    return pl.pallas_call(
        _es_body,
        grid=(EPAD // blk,),
        in_specs=[
            pl.BlockSpec((rows, D_EDGE), lambda j: (0, 0)),
            pl.BlockSpec((blk, D_EDGE), lambda j: (j, 0)),
        ],
        out_specs=pl.BlockSpec((rows, blk), lambda j: (0, j)),
        out_shape=jax.ShapeDtypeStruct((rows, EPAD), _f32),
    )(A_all, eap)


def _mid_body(o_ref, d_ref, w_ref, a_ref, h2_ref, s2_ref):
    parts = []
    for i in range(N_HEADS):
        agg = o_ref[i]
        den = jnp.sum(d_ref[i], axis=0)[:, None]
        hp = agg / (den + 1e-16)
        parts.append(_elu(_elu(hp)))
    x = jnp.concatenate(parts, axis=1)
    h2 = jnp.dot(x, w_ref[...], preferred_element_type=_f32)
    a = a_ref[...]
    a_l = a[:HID, 0:1]
    a_r = a[HID:2 * HID, 0:1]
    s_l = jnp.dot(h2, a_l, preferred_element_type=_f32)
    s_r = jnp.dot(h2, a_r, preferred_element_type=_f32)
    h2_ref[...] = h2
    s2_ref[...] = jnp.concatenate([s_l.T, s_r.T], axis=0)


def _tc_mid(out_l1, den_l1, W_out, a_out):
    blk = 1024
    return pl.pallas_call(
        _mid_body,
        grid=(NPAD // blk,),
        in_specs=[
            pl.BlockSpec((N_HEADS, blk, HID), lambda j: (0, j, 0)),
            pl.BlockSpec((N_HEADS, NS, blk), lambda j: (0, 0, j)),
            pl.BlockSpec((N_HEADS * HID, HID), lambda j: (0, 0)),
            pl.BlockSpec((2 * HID + D_EDGE, 1), lambda j: (0, 0)),
        ],
        out_specs=[
            pl.BlockSpec((blk, HID), lambda j: (j, 0)),
            pl.BlockSpec((2, blk), lambda j: (0, j)),
        ],
        out_shape=[
            jax.ShapeDtypeStruct((NPAD, HID), _f32),
            jax.ShapeDtypeStruct((2, NPAD), _f32),
        ],
    )(out_l1, den_l1, W_out, a_out)


def _final_body(p_ref, d_ref, o_ref):
    p = p_ref[0] + p_ref[1]
    den = jnp.sum(d_ref[0], axis=0)[:, None]
    hp = p / (den + 1e-16)
    o = _elu(hp)
    m = jnp.max(o, axis=1, keepdims=True)
    ls = jnp.log(jnp.sum(jnp.exp(o - m), axis=1, keepdims=True))
    o_ref[...] = o - m - ls


def _tc_final(partials, den_l2):
    blk = 1024
    return pl.pallas_call(
        _final_body,
        grid=(NPAD // blk,),
        in_specs=[
            pl.BlockSpec((NC, blk, HID), lambda j: (0, j, 0)),
            pl.BlockSpec((1, NC * NS, blk), lambda j: (0, 0, j)),
        ],
        out_specs=pl.BlockSpec((blk, HID), lambda j: (j, 0)),
        out_shape=jax.ShapeDtypeStruct((NPAD, HID), _f32),
    )(partials, den_l2)


_MESH = plsc.VectorSubcoreMesh(core_axis_name="c", subcore_axis_name="s")
_SC_PARAMS = pltpu.CompilerParams(needs_layout_passes=False)
_NEG = -3.0e38


def _iota16():
    return lax.iota(_i32, 16)


def _make_score_kernel(n_heads, hpc):
    k1_chunk = 512
    if hpc < n_heads:
        edges_per_tile = EPAD // NS
    else:
        edges_per_tile = EPAD // (NS * NC)
    n_chunks = edges_per_tile // k1_chunk
    groups = k1_chunk // 16

    def body(s_hbm, src_hbm, tgt_hbm, es_hbm, raw_hbm, tmax_hbm, *scr):
        tbls = scr[:2 * hpc]
        srcb, tgtb, esb, rawb, mbuf, tmaxb = scr[2 * hpc:]
        cid = lax.axis_index("c")
        sid = lax.axis_index("s")
        wid = cid * NS + sid
        head_base = cid * hpc if hpc < n_heads else 0
        if hpc < n_heads:
            edge_base = sid * edges_per_tile
        else:
            edge_base = wid * edges_per_tile

        for k in range(hpc):
            pltpu.sync_copy(s_hbm.at[head_base + k], tbls[k])
            pltpu.sync_copy(s_hbm.at[n_heads + head_base + k], tbls[hpc + k])
        for k in range(hpc):
            mbuf[k, :] = jnp.full((16,), _NEG, _f32)

        def chunk(ci, carry):
            base = pl.multiple_of(edge_base + ci * k1_chunk, k1_chunk)
            pltpu.sync_copy(src_hbm.at[pl.ds(base, k1_chunk)], srcb)
            pltpu.sync_copy(tgt_hbm.at[pl.ds(base, k1_chunk)], tgtb)
            for k in range(hpc):
                pltpu.sync_copy(es_hbm.at[head_base + k, pl.ds(base, k1_chunk)],
                                esb.at[k])
            for k in range(hpc):
                mk = mbuf[k, :]
                for g in range(groups):
                    si = srcb[pl.ds(g * 16, 16)]
                    ti = tgtb[pl.ds(g * 16, 16)]
                    sl = plsc.load_gather(tbls[k], [si])
                    sr = plsc.load_gather(tbls[hpc + k], [ti])
                    raw = sl + sr + esb[k, pl.ds(g * 16, 16)]
                    raw = jnp.where(raw >= 0, raw, 0.01 * raw)
                    eid = base + g * 16 + _iota16()
                    raw = jnp.where(eid < E_EDGES, raw, -1.0e30)
                    mk = jnp.maximum(mk, raw)
                    rawb[k, pl.ds(g * 16, 16)] = raw
                mbuf[k, :] = mk
                pltpu.sync_copy(rawb.at[k],
                                raw_hbm.at[head_base + k, pl.ds(base, k1_chunk)])
            return carry

        lax.fori_loop(0, n_chunks, chunk, 0)

        mv = jnp.full((16,), _NEG, _f32)
        for k in range(hpc):
            msc = jnp.max(mbuf[k, :])
            lane = head_base + k
            mv = jnp.where(_iota16() == lane, msc, mv)
        tmaxb[...] = mv
        pltpu.sync_copy(tmaxb, tmax_hbm.at[wid])

    return pl.kernel(
        body,
        out_type=[
            jax.ShapeDtypeStruct((n_heads, EPAD), _f32),
            jax.ShapeDtypeStruct((NC * NS, 16), _f32),
        ],
        mesh=_MESH,
        compiler_params=_SC_PARAMS,
        scratch_types=[
            *[pltpu.VMEM((NPAD,), _f32) for _ in range(2 * hpc)],
            pltpu.VMEM((k1_chunk,), _i32),
            pltpu.VMEM((k1_chunk,), _i32),
            pltpu.VMEM((hpc, k1_chunk), _f32),
            pltpu.VMEM((hpc, k1_chunk), _f32),
            pltpu.VMEM((hpc, 16), _f32),
            pltpu.VMEM((16,), _f32),
        ],
    )


def _make_agg_kernel(n_heads, hpc):
    if hpc < n_heads:
        chunks_per_tile = (EPAD // NS) // CHUNK
        out_rows = n_heads
        den_tiles = NS
    else:
        chunks_per_tile = (EPAD // (NS * NC)) // CHUNK
        out_rows = NC
        den_tiles = NC * NS
    nslice = NPAD // NS
    ncopies = nslice // CHUNK

    def body(h_hbm, src_hbm, tgt_hbm, raw_hbm, tmax_hbm, out_hbm, den_hbm,
             acc, srcb, tgtb, rawcb, rows, outb, denb, tmr, gsem):
        cid = lax.axis_index("c")
        sid = lax.axis_index("s")
        head_base = cid * hpc if hpc < n_heads else 0
        if hpc < n_heads:
            edge_base = sid * (EPAD // NS)
            den_slot = sid
        else:
            edge_base = (cid * NS + sid) * (EPAD // (NS * NC))
            den_slot = cid * NS + sid

        pltpu.sync_copy(tmax_hbm, tmr)
        mv = jnp.full((16,), _NEG, _f32)
        for r in range(NC * NS):
            mv = jnp.maximum(mv, tmr[r, :])

        def zrow(ci, carry):
            z = jnp.zeros((16,), _f32)
            for j in range(HID // 16):
                outb[ci, pl.ds(j * 16, 16)] = z
            return carry
        lax.fori_loop(0, CHUNK, zrow, 0)

        def zden(ci, carry):
            denb[pl.ds(ci * 16, 16)] = jnp.zeros((16,), _f32)
            return carry

        for k in range(hpc):
            head = head_base + k
            for q in range(ncopies):
                pltpu.sync_copy(outb, acc.at[pl.ds(sid * nslice + q * CHUNK,
                                                   CHUNK)])
            lax.fori_loop(0, NPAD // 16, zden, 0)
            plsc.subcore_barrier()

            msc = jnp.max(jnp.where(_iota16() == head, mv, _NEG))

            def chunk(ci, carry):
                base = pl.multiple_of(edge_base + ci * CHUNK, CHUNK)
                pltpu.sync_copy(src_hbm.at[pl.ds(base, CHUNK)], srcb)
                pltpu.sync_copy(tgt_hbm.at[pl.ds(base, CHUNK)], tgtb)
                pltpu.sync_copy(raw_hbm.at[head, pl.ds(base, CHUNK)], rawcb)
                pltpu.async_copy(h_hbm.at[head].at[tgtb], rows, gsem).wait()
                for g in range(CHUNK // 16):
                    r16 = rawcb[pl.ds(g * 16, 16)]
                    w16 = jnp.exp(r16 - msc)
                    si16 = srcb[pl.ds(g * 16, 16)]
                    plsc.addupdate_scatter(denb, [si16], w16)
                    for c in range(16):
                        wsc = w16[c]
                        rr = g * 16 + c
                        for j in range(HID // 16):
                            outb[rr, pl.ds(j * 16, 16)] = (
                                rows[rr, pl.ds(j * 16, 16)] * wsc)
                pltpu.sync_copy(outb, acc.at[srcb], add=True)
                return carry

            lax.fori_loop(0, chunks_per_tile, chunk, 0)
            plsc.subcore_barrier()

            out_row = head if hpc < n_heads else cid
            for q in range(ncopies):
                rr = sid * nslice + q * CHUNK
                pltpu.sync_copy(acc.at[pl.ds(rr, CHUNK)],
                                out_hbm.at[out_row, pl.ds(rr, CHUNK)])
            pltpu.sync_copy(denb, den_hbm.at[head, den_slot])
            if k + 1 < hpc:
                lax.fori_loop(0, CHUNK, zrow, 0)

    return pl.kernel(
        body,
        out_type=[
            jax.ShapeDtypeStruct((out_rows, NPAD, HID), _f32),
            jax.ShapeDtypeStruct((n_heads, den_tiles, NPAD), _f32),
        ],
        mesh=_MESH,
        compiler_params=_SC_PARAMS,
        scratch_types=[
            pltpu.VMEM_SHARED((NPAD, HID), _f32),
            pltpu.VMEM((CHUNK,), _i32),
            pltpu.VMEM((CHUNK,), _i32),
            pltpu.VMEM((CHUNK,), _f32),
            pltpu.VMEM((CHUNK, HID), _f32),
            pltpu.VMEM((CHUNK, HID), _f32),
            pltpu.VMEM((NPAD,), _f32),
            pltpu.VMEM((NC * NS, 16), _f32),
            pltpu.SemaphoreType.DMA,
        ],
    )


_score_l1 = _make_score_kernel(N_HEADS, N_HEADS // NC)
_score_l2 = _make_score_kernel(1, 1)
_agg_l1 = _make_agg_kernel(N_HEADS, N_HEADS // NC)
_agg_l2 = _make_agg_kernel(1, 1)


@jax.jit
def _run(X, edge_index, edge_attr, W_heads, a_heads, W_out, a_out):
    Xp = jnp.pad(X, ((0, NPAD - N_NODES), (0, 0)))
    src = jnp.pad(edge_index[0], (0, EPAD - E_EDGES))
    tgt = jnp.pad(edge_index[1], (0, EPAD - E_EDGES))
    eap = jnp.pad(edge_attr, ((0, EPAD - E_EDGES), (0, 0)))

    H1 = _tc_heads_matmul(Xp, W_heads)
    S1 = _tc_node_scores(Xp, W_heads, a_heads)
    A_all = jnp.concatenate(
        [a_heads[:, 2 * HID:, 0], a_out[2 * HID:, 0][None]], axis=0)
    Es_all = _tc_edge_scores(A_all, eap)
    Es1 = Es_all[:N_HEADS]
    Es2 = Es_all[N_HEADS:]

    raw1, tmax1 = _score_l1(S1, src, tgt, Es1)
    out_l1, den_l1 = _agg_l1(H1, src, tgt, raw1, tmax1)

    H2, S2 = _tc_mid(out_l1, den_l1, W_out, a_out)

    raw2, tmax2 = _score_l2(S2, src, tgt, Es2)
    partials, den_l2 = _agg_l2(H2[None], src, tgt, raw2, tmax2)

    out = _tc_final(partials, den_l2)
    return out[:N_NODES]


def kernel(X, edge_index, edge_attr, W_heads, a_heads, W_out, a_out):
    return _run(X, edge_index, edge_attr, W_heads, a_heads, W_out, a_out)


# speedup vs baseline: 8.0860x; 1.4252x over previous
import functools

import jax
import jax.numpy as jnp
from jax import lax
from jax.experimental import pallas as pl
from jax.experimental.pallas import tpu as pltpu
from jax.experimental.pallas import tpu_sc as plsc

N_NODES = 10000
NPAD = 10240
E_EDGES = 320000
EPAD = 327680
D_IN = 128
HID = 128
D_EDGE = 16
N_HEADS = 8
NS = 16
NC = 2
ACC_W = 144
CHUNK = 64
SUP = 4
EROWS = EPAD // CHUNK

_f32 = jnp.float32
_i32 = jnp.int32


def _elu(x):
    return jnp.where(x > 0, x, jnp.exp(jnp.minimum(x, 0.0)) - 1.0)


def _h_body(x_ref, w_ref, h_ref):
    h_ref[0] = jnp.dot(x_ref[...], w_ref[0], preferred_element_type=_f32)


def _tc_heads_matmul(Xp, W_heads):
    blk = 2048
    return pl.pallas_call(
        _h_body,
        grid=(N_HEADS, NPAD // blk),
        in_specs=[
            pl.BlockSpec((blk, D_IN), lambda i, j: (j, 0)),
            pl.BlockSpec((1, D_IN, HID), lambda i, j: (i, 0, 0)),
        ],
        out_specs=pl.BlockSpec((1, blk, HID), lambda i, j: (i, j, 0)),
        out_shape=jax.ShapeDtypeStruct((N_HEADS, NPAD, HID), _f32),
    )(Xp, W_heads)


def _s_body(x_ref, w_ref, a_ref, s_ref):
    W = w_ref[...]
    a = a_ref[...]
    a_l = a[:, :HID, 0]
    a_r = a[:, HID:2 * HID, 0]
    q_l = lax.dot_general(W, a_l, (((2,), (1,)), ((0,), (0,))),
                          preferred_element_type=_f32)
    q_r = lax.dot_general(W, a_r, (((2,), (1,)), ((0,), (0,))),
                          preferred_element_type=_f32)
    q = jnp.concatenate([q_l, q_r], axis=0)
    s_ref[...] = lax.dot_general(q, x_ref[...], (((1,), (1,)), ((), ())),
                                 preferred_element_type=_f32)


def _tc_node_scores(Xp, W_heads, a_heads):
    blk = 2048
    return pl.pallas_call(
        _s_body,
        grid=(NPAD // blk,),
        in_specs=[
            pl.BlockSpec((blk, D_IN), lambda j: (j, 0)),
            pl.BlockSpec((N_HEADS, D_IN, HID), lambda j: (0, 0, 0)),
            pl.BlockSpec((N_HEADS, 2 * HID + D_EDGE, 1), lambda j: (0, 0, 0)),
        ],
        out_specs=pl.BlockSpec((2 * N_HEADS, blk), lambda j: (0, j)),
        out_shape=jax.ShapeDtypeStruct((2 * N_HEADS, NPAD), _f32),
    )(Xp, W_heads, a_heads)


def _es_body(a_ref, e_ref, o_ref):
    o_ref[...] = lax.dot_general(a_ref[...], e_ref[...],
                                 (((1,), (1,)), ((), ())),
                                 preferred_element_type=_f32)


def _tc_edge_scores(A_all, eap):
    rows = A_all.shape[0]
    blk = 8192
    return pl.pallas_call(
        _es_body,
        grid=(EPAD // blk,),
        in_specs=[
            pl.BlockSpec((rows, D_EDGE), lambda j: (0, 0)),
            pl.BlockSpec((blk, D_EDGE), lambda j: (j, 0)),
        ],
        out_specs=pl.BlockSpec((rows, blk), lambda j: (0, j)),
        out_shape=jax.ShapeDtypeStruct((rows, EPAD), _f32),
    )(A_all, eap)


def _mid_body(o_ref, d_ref, w_ref, a_ref, h2_ref, s2_ref):
    parts = []
    for i in range(N_HEADS):
        agg = o_ref[i]
        den = jnp.sum(d_ref[i], axis=0)[:, None]
        hp = agg / (den + 1e-16)
        parts.append(_elu(_elu(hp)))
    x = jnp.concatenate(parts, axis=1)
    h2 = jnp.dot(x, w_ref[...], preferred_element_type=_f32)
    a = a_ref[...]
    a_l = a[:HID, 0:1]
    a_r = a[HID:2 * HID, 0:1]
    s_l = jnp.dot(h2, a_l, preferred_element_type=_f32)
    s_r = jnp.dot(h2, a_r, preferred_element_type=_f32)
    h2_ref[...] = h2
    s2_ref[...] = jnp.concatenate([s_l.T, s_r.T], axis=0)


def _tc_mid(out_l1, den_l1, W_out, a_out):
    blk = 1024
    return pl.pallas_call(
        _mid_body,
        grid=(NPAD // blk,),
        in_specs=[
            pl.BlockSpec((N_HEADS, blk, HID), lambda j: (0, j, 0)),
            pl.BlockSpec((N_HEADS, NS, blk), lambda j: (0, 0, j)),
            pl.BlockSpec((N_HEADS * HID, HID), lambda j: (0, 0)),
            pl.BlockSpec((2 * HID + D_EDGE, 1), lambda j: (0, 0)),
        ],
        out_specs=[
            pl.BlockSpec((blk, HID), lambda j: (j, 0)),
            pl.BlockSpec((2, blk), lambda j: (0, j)),
        ],
        out_shape=[
            jax.ShapeDtypeStruct((NPAD, HID), _f32),
            jax.ShapeDtypeStruct((2, NPAD), _f32),
        ],
    )(out_l1, den_l1, W_out, a_out)


def _final_body(p_ref, d_ref, o_ref):
    p = p_ref[0] + p_ref[1]
    den = jnp.sum(d_ref[0], axis=0)[:, None]
    hp = p / (den + 1e-16)
    o = _elu(hp)
    m = jnp.max(o, axis=1, keepdims=True)
    ls = jnp.log(jnp.sum(jnp.exp(o - m), axis=1, keepdims=True))
    o_ref[...] = o - m - ls


def _tc_final(partials, den_l2):
    blk = 1024
    return pl.pallas_call(
        _final_body,
        grid=(NPAD // blk,),
        in_specs=[
            pl.BlockSpec((NC, blk, HID), lambda j: (0, j, 0)),
            pl.BlockSpec((1, NC * NS, blk), lambda j: (0, 0, j)),
        ],
        out_specs=pl.BlockSpec((blk, HID), lambda j: (j, 0)),
        out_shape=jax.ShapeDtypeStruct((NPAD, HID), _f32),
    )(partials, den_l2)


_MESH = plsc.VectorSubcoreMesh(core_axis_name="c", subcore_axis_name="s")
_SC_PARAMS = pltpu.CompilerParams(needs_layout_passes=False)
_NEG = -3.0e38


def _iota16():
    return lax.iota(_i32, 16)


def _make_score_kernel(n_heads, hpc):
    k1_chunk = 512
    if hpc < n_heads:
        edges_per_tile = EPAD // NS
    else:
        edges_per_tile = EPAD // (NS * NC)
    n_chunks = edges_per_tile // k1_chunk
    groups = k1_chunk // 16

    def body(s_hbm, src_hbm, tgt_hbm, es_hbm, raw_hbm, tmax_hbm, *scr):
        tbls = scr[:2 * hpc]
        srcb, tgtb, esb, rawb, mbuf, tmaxb = scr[2 * hpc:]
        cid = lax.axis_index("c")
        sid = lax.axis_index("s")
        wid = cid * NS + sid
        head_base = cid * hpc if hpc < n_heads else 0
        if hpc < n_heads:
            edge_base = sid * edges_per_tile
        else:
            edge_base = wid * edges_per_tile

        for k in range(hpc):
            pltpu.sync_copy(s_hbm.at[head_base + k], tbls[k])
            pltpu.sync_copy(s_hbm.at[n_heads + head_base + k], tbls[hpc + k])
        for k in range(hpc):
            mbuf[k, :] = jnp.full((16,), _NEG, _f32)

        def chunk(ci, carry):
            base = pl.multiple_of(edge_base + ci * k1_chunk, k1_chunk)
            pltpu.sync_copy(src_hbm.at[pl.ds(base, k1_chunk)], srcb)
            pltpu.sync_copy(tgt_hbm.at[pl.ds(base, k1_chunk)], tgtb)
            for k in range(hpc):
                pltpu.sync_copy(es_hbm.at[head_base + k, pl.ds(base, k1_chunk)],
                                esb.at[k])
            for k in range(hpc):
                mk = mbuf[k, :]
                for g in range(groups):
                    si = srcb[pl.ds(g * 16, 16)]
                    ti = tgtb[pl.ds(g * 16, 16)]
                    sl = plsc.load_gather(tbls[k], [si])
                    sr = plsc.load_gather(tbls[hpc + k], [ti])
                    raw = sl + sr + esb[k, pl.ds(g * 16, 16)]
                    raw = jnp.where(raw >= 0, raw, 0.01 * raw)
                    eid = base + g * 16 + _iota16()
                    raw = jnp.where(eid < E_EDGES, raw, -1.0e30)
                    mk = jnp.maximum(mk, raw)
                    rawb[k, pl.ds(g * 16, 16)] = raw
                mbuf[k, :] = mk
                pltpu.sync_copy(rawb.at[k],
                                raw_hbm.at[head_base + k, pl.ds(base, k1_chunk)])
            return carry

        lax.fori_loop(0, n_chunks, chunk, 0)

        mv = jnp.full((16,), _NEG, _f32)
        for k in range(hpc):
            msc = jnp.max(mbuf[k, :])
            lane = head_base + k
            mv = jnp.where(_iota16() == lane, msc, mv)
        tmaxb[...] = mv
        pltpu.sync_copy(tmaxb, tmax_hbm.at[wid])

    return pl.kernel(
        body,
        out_type=[
            jax.ShapeDtypeStruct((n_heads, EPAD), _f32),
            jax.ShapeDtypeStruct((NC * NS, 16), _f32),
        ],
        mesh=_MESH,
        compiler_params=_SC_PARAMS,
        scratch_types=[
            *[pltpu.VMEM((NPAD,), _f32) for _ in range(2 * hpc)],
            pltpu.VMEM((k1_chunk,), _i32),
            pltpu.VMEM((k1_chunk,), _i32),
            pltpu.VMEM((hpc, k1_chunk), _f32),
            pltpu.VMEM((hpc, k1_chunk), _f32),
            pltpu.VMEM((hpc, 16), _f32),
            pltpu.VMEM((16,), _f32),
        ],
    )


def _make_agg_kernel(n_heads, hpc):
    if hpc < n_heads:
        chunks_per_tile = (EPAD // NS) // CHUNK
        out_rows = n_heads
        den_tiles = NS
    else:
        chunks_per_tile = (EPAD // (NS * NC)) // CHUNK
        out_rows = NC
        den_tiles = NC * NS
    nslice = NPAD // NS
    ncopies = nslice // CHUNK

    def body(h_hbm, src_hbm, tgt_hbm, raw_hbm, tmax_hbm, out_hbm, den_hbm,
             acc, srcb0, srcb1, tgtb0, tgtb1, rawb0, rawb1, scb0, scb1,
             rows0, rows1, outb0, outb1, denb, tmr,
             gsem0, gsem1, ssem0, ssem1, isem0, isem1):
        srcb = (srcb0, srcb1)
        tgtb = (tgtb0, tgtb1)
        rawb = (rawb0, rawb1)
        scb = (scb0, scb1)
        rows = (rows0, rows1)
        outb = (outb0, outb1)
        gsem = (gsem0, gsem1)
        ssem = (ssem0, ssem1)
        isem = (isem0, isem1)
        cid = lax.axis_index("c")
        sid = lax.axis_index("s")
        head_base = cid * hpc if hpc < n_heads else 0
        if hpc < n_heads:
            edge_base = sid * (EPAD // NS)
            den_slot = sid
        else:
            edge_base = (cid * NS + sid) * (EPAD // (NS * NC))
            den_slot = cid * NS + sid
        row_base = edge_base // CHUNK
        nch = chunks_per_tile

        pltpu.sync_copy(tmax_hbm, tmr)
        mv = jnp.full((16,), _NEG, _f32)
        for r in range(NC * NS):
            mv = jnp.maximum(mv, tmr[r, :])

        def zrow(ci, carry):
            z = jnp.zeros((16,), _f32)
            for j in range(HID // 16):
                outb0[ci, pl.ds(j * 16, 16)] = z
            return carry

        def zden(ci, carry):
            denb[pl.ds(ci * 16, 16)] = jnp.zeros((16,), _f32)
            return carry

        z16i = jnp.zeros((16,), _i32)
        for j in range(CHUNK // 16):
            scb0[pl.ds(j * 16, 16)] = z16i
            scb1[pl.ds(j * 16, 16)] = z16i

        for k in range(hpc):
            head = head_base + k
            lax.fori_loop(0, CHUNK, zrow, 0)
            for q in range(ncopies):
                pltpu.sync_copy(outb0, acc.at[pl.ds(sid * nslice + q * CHUNK,
                                                    CHUNK)])
            lax.fori_loop(0, NPAD // 16, zden, 0)
            plsc.subcore_barrier()

            msc = jnp.max(jnp.where(_iota16() == head, mv, _NEG))

            def clamp(cc):
                return jnp.minimum(row_base + cc, row_base + nch - 1)

            def idx_start(cc, p):
                rb = clamp(cc)
                pltpu.async_copy(src_hbm.at[rb], srcb[p], isem[p])
                pltpu.async_copy(tgt_hbm.at[rb], tgtb[p], isem[p])
                pltpu.async_copy(raw_hbm.at[head, rb], rawb[p], isem[p])

            def idx_wait(cc, p):
                rb = clamp(cc)
                pltpu.make_async_copy(src_hbm.at[rb], srcb[p], isem[p]).wait()
                pltpu.make_async_copy(tgt_hbm.at[rb], tgtb[p], isem[p]).wait()
                pltpu.make_async_copy(raw_hbm.at[head, rb], rawb[p],
                                      isem[p]).wait()

            def gather_start(p):
                pltpu.async_copy(h_hbm.at[head].at[tgtb[p]], rows[p], gsem[p])

            def gather_wait(p):
                pltpu.make_async_copy(h_hbm.at[head].at[tgtb[p]], rows[p],
                                      gsem[p]).wait()

            def scatter_start(p):
                pltpu.async_copy(outb[p], acc.at[scb[p]], ssem[p], add=True)

            def scatter_wait(p):
                pltpu.make_async_copy(outb[p], acc.at[scb[p]], ssem[p]).wait()

            def compute(p):
                def group(g, carry):
                    gofs = pl.multiple_of(g * 16, 16)
                    r16 = rawb[p][pl.ds(gofs, 16)]
                    w16 = jnp.exp(r16 - msc)
                    si16 = srcb[p][pl.ds(gofs, 16)]
                    plsc.addupdate_scatter(denb, [si16], w16)
                    scb[p][pl.ds(gofs, 16)] = si16
                    for c2 in range(16):
                        wsc = w16[c2]
                        rr = gofs + c2
                        for j in range(HID // 16):
                            outb[p][rr, pl.ds(j * 16, 16)] = (
                                rows[p][rr, pl.ds(j * 16, 16)] * wsc)
                    return carry
                lax.fori_loop(0, CHUNK // 16, group, 0)

            def step(c, p):
                gather_wait(p)
                idx_wait(c + 1, 1 - p)
                gather_start(1 - p)
                scatter_wait(p)
                compute(p)
                scatter_start(p)
                idx_start(c + 2, p)

            idx_start(0, 0)
            idx_start(1, 1)
            idx_wait(0, 0)
            gather_start(0)
            pltpu.async_copy(outb0, acc.at[scb0], ssem0, add=True)
            pltpu.async_copy(outb0, acc.at[scb1], ssem1, add=True)

            def pair(m, carry):
                step(2 * m, 0)
                step(2 * m + 1, 1)
                return carry

            lax.fori_loop(0, nch // 2, pair, 0)
            scatter_wait(0)
            scatter_wait(1)
            gather_wait(0)
            idx_wait(nch - 1, 1)
            plsc.subcore_barrier()

            out_row = head if hpc < n_heads else cid
            for q in range(ncopies):
                rr = sid * nslice + q * CHUNK
                pltpu.sync_copy(acc.at[pl.ds(rr, CHUNK)],
                                out_hbm.at[out_row, pl.ds(rr, CHUNK)])
            pltpu.sync_copy(denb, den_hbm.at[head, den_slot])

    return pl.kernel(
        body,
        out_type=[
            jax.ShapeDtypeStruct((out_rows, NPAD, HID), _f32),
            jax.ShapeDtypeStruct((n_heads, den_tiles, NPAD), _f32),
        ],
        mesh=_MESH,
        compiler_params=_SC_PARAMS,
        scratch_types=[
            pltpu.VMEM_SHARED((NPAD, HID), _f32),
            pltpu.VMEM((CHUNK,), _i32),
            pltpu.VMEM((CHUNK,), _i32),
            pltpu.VMEM((CHUNK,), _i32),
            pltpu.VMEM((CHUNK,), _i32),
            pltpu.VMEM((CHUNK,), _f32),
            pltpu.VMEM((CHUNK,), _f32),
            pltpu.VMEM((CHUNK,), _i32),
            pltpu.VMEM((CHUNK,), _i32),
            pltpu.VMEM((CHUNK, HID), _f32),
            pltpu.VMEM((CHUNK, HID), _f32),
            pltpu.VMEM((CHUNK, HID), _f32),
            pltpu.VMEM((CHUNK, HID), _f32),
            pltpu.VMEM((NPAD,), _f32),
            pltpu.VMEM((NC * NS, 16), _f32),
            pltpu.SemaphoreType.DMA,
            pltpu.SemaphoreType.DMA,
            pltpu.SemaphoreType.DMA,
            pltpu.SemaphoreType.DMA,
            pltpu.SemaphoreType.DMA,
            pltpu.SemaphoreType.DMA,
        ],
    )


_score_l1 = _make_score_kernel(N_HEADS, N_HEADS // NC)
_score_l2 = _make_score_kernel(1, 1)
_agg_l1 = _make_agg_kernel(N_HEADS, N_HEADS // NC)
_agg_l2 = _make_agg_kernel(1, 1)


@jax.jit
def _run(X, edge_index, edge_attr, W_heads, a_heads, W_out, a_out):
    Xp = jnp.pad(X, ((0, NPAD - N_NODES), (0, 0)))
    src = jnp.pad(edge_index[0], (0, EPAD - E_EDGES))
    tgt = jnp.pad(edge_index[1], (0, EPAD - E_EDGES))
    eap = jnp.pad(edge_attr, ((0, EPAD - E_EDGES), (0, 0)))

    H1 = _tc_heads_matmul(Xp, W_heads)
    S1 = _tc_node_scores(Xp, W_heads, a_heads)
    A_all = jnp.concatenate(
        [a_heads[:, 2 * HID:, 0], a_out[2 * HID:, 0][None]], axis=0)
    Es_all = _tc_edge_scores(A_all, eap)
    Es1 = Es_all[:N_HEADS]
    Es2 = Es_all[N_HEADS:]

    raw1, tmax1 = _score_l1(S1, src, tgt, Es1)
    src2d = src.reshape(EROWS, CHUNK)
    tgt2d = tgt.reshape(EROWS, CHUNK)
    out_l1, den_l1 = _agg_l1(H1, src2d, tgt2d,
                             raw1.reshape(N_HEADS, EROWS, CHUNK), tmax1)

    H2, S2 = _tc_mid(out_l1, den_l1, W_out, a_out)

    raw2, tmax2 = _score_l2(S2, src, tgt, Es2)
    partials, den_l2 = _agg_l2(H2[None], src2d, tgt2d,
                               raw2.reshape(1, EROWS, CHUNK), tmax2)

    out = _tc_final(partials, den_l2)
    return out[:N_NODES]


def kernel(X, edge_index, edge_attr, W_heads, a_heads, W_out, a_out):
    return _run(X, edge_index, edge_attr, W_heads, a_heads, W_out, a_out)


# speedup vs baseline: 8.1218x; 1.0044x over previous
import functools

import jax
import jax.numpy as jnp
from jax import lax
from jax.experimental import pallas as pl
from jax.experimental.pallas import tpu as pltpu
from jax.experimental.pallas import tpu_sc as plsc

N_NODES = 10000
NPAD = 10240
E_EDGES = 320000
EPAD = 327680
D_IN = 128
HID = 128
D_EDGE = 16
N_HEADS = 8
NS = 16
NC = 2
ACC_W = 144
CHUNK = 64
ACCN = 10112
SUP = 4
EROWS = EPAD // CHUNK

_f32 = jnp.float32
_i32 = jnp.int32


def _elu(x):
    return jnp.where(x > 0, x, jnp.exp(jnp.minimum(x, 0.0)) - 1.0)


def _h_body(x_ref, w_ref, h_ref):
    h_ref[0] = jnp.dot(x_ref[...], w_ref[0], preferred_element_type=_f32)


def _tc_heads_matmul(Xp, W_heads):
    blk = 2048
    return pl.pallas_call(
        _h_body,
        grid=(N_HEADS, NPAD // blk),
        in_specs=[
            pl.BlockSpec((blk, D_IN), lambda i, j: (j, 0)),
            pl.BlockSpec((1, D_IN, HID), lambda i, j: (i, 0, 0)),
        ],
        out_specs=pl.BlockSpec((1, blk, HID), lambda i, j: (i, j, 0)),
        out_shape=jax.ShapeDtypeStruct((N_HEADS, NPAD, HID), _f32),
    )(Xp, W_heads)


def _s_body(x_ref, w_ref, a_ref, s_ref):
    W = w_ref[...]
    a = a_ref[...]
    a_l = a[:, :HID, 0]
    a_r = a[:, HID:2 * HID, 0]
    q_l = lax.dot_general(W, a_l, (((2,), (1,)), ((0,), (0,))),
                          preferred_element_type=_f32)
    q_r = lax.dot_general(W, a_r, (((2,), (1,)), ((0,), (0,))),
                          preferred_element_type=_f32)
    q = jnp.concatenate([q_l, q_r], axis=0)
    s_ref[...] = lax.dot_general(q, x_ref[...], (((1,), (1,)), ((), ())),
                                 preferred_element_type=_f32)


def _tc_node_scores(Xp, W_heads, a_heads):
    blk = 2048
    return pl.pallas_call(
        _s_body,
        grid=(NPAD // blk,),
        in_specs=[
            pl.BlockSpec((blk, D_IN), lambda j: (j, 0)),
            pl.BlockSpec((N_HEADS, D_IN, HID), lambda j: (0, 0, 0)),
            pl.BlockSpec((N_HEADS, 2 * HID + D_EDGE, 1), lambda j: (0, 0, 0)),
        ],
        out_specs=pl.BlockSpec((2 * N_HEADS, blk), lambda j: (0, j)),
        out_shape=jax.ShapeDtypeStruct((2 * N_HEADS, NPAD), _f32),
    )(Xp, W_heads, a_heads)


def _es_body(a_ref, e_ref, o_ref):
    o_ref[...] = lax.dot_general(a_ref[...], e_ref[...],
                                 (((1,), (1,)), ((), ())),
                                 preferred_element_type=_f32)


def _tc_edge_scores(A_all, eap):
    rows = A_all.shape[0]
    blk = 8192
    return pl.pallas_call(
        _es_body,
        grid=(EPAD // blk,),
        in_specs=[
            pl.BlockSpec((rows, D_EDGE), lambda j: (0, 0)),
            pl.BlockSpec((blk, D_EDGE), lambda j: (j, 0)),
        ],
        out_specs=pl.BlockSpec((rows, blk), lambda j: (0, j)),
        out_shape=jax.ShapeDtypeStruct((rows, EPAD), _f32),
    )(A_all, eap)


def _mid_body(o_ref, d_ref, w_ref, a_ref, h2_ref, s2_ref):
    parts = []
    for i in range(N_HEADS):
        agg = o_ref[i]
        den = jnp.sum(d_ref[i], axis=0)[:, None]
        hp = agg / (den + 1e-16)
        parts.append(_elu(_elu(hp)))
    x = jnp.concatenate(parts, axis=1)
    h2 = jnp.dot(x, w_ref[...], preferred_element_type=_f32)
    a = a_ref[...]
    a_l = a[:HID, 0:1]
    a_r = a[HID:2 * HID, 0:1]
    s_l = jnp.dot(h2, a_l, preferred_element_type=_f32)
    s_r = jnp.dot(h2, a_r, preferred_element_type=_f32)
    h2_ref[...] = h2
    s2_ref[...] = jnp.concatenate([s_l.T, s_r.T], axis=0)


def _tc_mid(out_l1, den_l1, W_out, a_out):
    blk = 1024
    return pl.pallas_call(
        _mid_body,
        grid=(NPAD // blk,),
        in_specs=[
            pl.BlockSpec((N_HEADS, blk, HID), lambda j: (0, j, 0)),
            pl.BlockSpec((N_HEADS, NS, blk), lambda j: (0, 0, j)),
            pl.BlockSpec((N_HEADS * HID, HID), lambda j: (0, 0)),
            pl.BlockSpec((2 * HID + D_EDGE, 1), lambda j: (0, 0)),
        ],
        out_specs=[
            pl.BlockSpec((blk, HID), lambda j: (j, 0)),
            pl.BlockSpec((2, blk), lambda j: (0, j)),
        ],
        out_shape=[
            jax.ShapeDtypeStruct((NPAD, HID), _f32),
            jax.ShapeDtypeStruct((2, NPAD), _f32),
        ],
    )(out_l1, den_l1, W_out, a_out)


def _final_body(p_ref, d_ref, o_ref):
    p = p_ref[0] + p_ref[1]
    den = jnp.sum(d_ref[0], axis=0)[:, None]
    hp = p / (den + 1e-16)
    o = _elu(hp)
    m = jnp.max(o, axis=1, keepdims=True)
    ls = jnp.log(jnp.sum(jnp.exp(o - m), axis=1, keepdims=True))
    o_ref[...] = o - m - ls


def _tc_final(partials, den_l2):
    blk = 1024
    return pl.pallas_call(
        _final_body,
        grid=(NPAD // blk,),
        in_specs=[
            pl.BlockSpec((NC, blk, HID), lambda j: (0, j, 0)),
            pl.BlockSpec((1, NC * NS, blk), lambda j: (0, 0, j)),
        ],
        out_specs=pl.BlockSpec((blk, HID), lambda j: (j, 0)),
        out_shape=jax.ShapeDtypeStruct((NPAD, HID), _f32),
    )(partials, den_l2)


_MESH = plsc.VectorSubcoreMesh(core_axis_name="c", subcore_axis_name="s")
_SC_PARAMS = pltpu.CompilerParams(needs_layout_passes=False)
_NEG = -3.0e38


def _iota16():
    return lax.iota(_i32, 16)


def _make_score_kernel(n_heads, hpc):
    k1_chunk = 512
    if hpc < n_heads:
        edges_per_tile = EPAD // NS
    else:
        edges_per_tile = EPAD // (NS * NC)
    n_chunks = edges_per_tile // k1_chunk
    groups = k1_chunk // 16

    def body(s_hbm, src_hbm, tgt_hbm, es_hbm, raw_hbm, tmax_hbm, *scr):
        tbls = scr[:2 * hpc]
        srcb, tgtb, esb, rawb, mbuf, tmaxb = scr[2 * hpc:]
        cid = lax.axis_index("c")
        sid = lax.axis_index("s")
        wid = cid * NS + sid
        head_base = cid * hpc if hpc < n_heads else 0
        if hpc < n_heads:
            edge_base = sid * edges_per_tile
        else:
            edge_base = wid * edges_per_tile

        for k in range(hpc):
            pltpu.sync_copy(s_hbm.at[head_base + k], tbls[k])
            pltpu.sync_copy(s_hbm.at[n_heads + head_base + k], tbls[hpc + k])
        for k in range(hpc):
            mbuf[k, :] = jnp.full((16,), _NEG, _f32)

        def chunk(ci, carry):
            base = pl.multiple_of(edge_base + ci * k1_chunk, k1_chunk)
            pltpu.sync_copy(src_hbm.at[pl.ds(base, k1_chunk)], srcb)
            pltpu.sync_copy(tgt_hbm.at[pl.ds(base, k1_chunk)], tgtb)
            for k in range(hpc):
                pltpu.sync_copy(es_hbm.at[head_base + k, pl.ds(base, k1_chunk)],
                                esb.at[k])
            for k in range(hpc):
                mk = mbuf[k, :]
                for g in range(groups):
                    si = srcb[pl.ds(g * 16, 16)]
                    ti = tgtb[pl.ds(g * 16, 16)]
                    sl = plsc.load_gather(tbls[k], [si])
                    sr = plsc.load_gather(tbls[hpc + k], [ti])
                    raw = sl + sr + esb[k, pl.ds(g * 16, 16)]
                    raw = jnp.where(raw >= 0, raw, 0.01 * raw)
                    eid = base + g * 16 + _iota16()
                    raw = jnp.where(eid < E_EDGES, raw, -1.0e30)
                    mk = jnp.maximum(mk, raw)
                    rawb[k, pl.ds(g * 16, 16)] = raw
                mbuf[k, :] = mk
                pltpu.sync_copy(rawb.at[k],
                                raw_hbm.at[head_base + k, pl.ds(base, k1_chunk)])
            return carry

        lax.fori_loop(0, n_chunks, chunk, 0)

        mv = jnp.full((16,), _NEG, _f32)
        for k in range(hpc):
            msc = jnp.max(mbuf[k, :])
            lane = head_base + k
            mv = jnp.where(_iota16() == lane, msc, mv)
        tmaxb[...] = mv
        pltpu.sync_copy(tmaxb, tmax_hbm.at[wid])

    return pl.kernel(
        body,
        out_type=[
            jax.ShapeDtypeStruct((n_heads, EPAD), _f32),
            jax.ShapeDtypeStruct((NC * NS, 16), _f32),
        ],
        mesh=_MESH,
        compiler_params=_SC_PARAMS,
        scratch_types=[
            *[pltpu.VMEM((NPAD,), _f32) for _ in range(2 * hpc)],
            pltpu.VMEM((k1_chunk,), _i32),
            pltpu.VMEM((k1_chunk,), _i32),
            pltpu.VMEM((hpc, k1_chunk), _f32),
            pltpu.VMEM((hpc, k1_chunk), _f32),
            pltpu.VMEM((hpc, 16), _f32),
            pltpu.VMEM((16,), _f32),
        ],
    )


def _make_agg_kernel(n_heads, hpc):
    if hpc < n_heads:
        chunks_per_tile = (EPAD // NS) // CHUNK
        out_rows = n_heads
        den_tiles = NS
    else:
        chunks_per_tile = (EPAD // (NS * NC)) // CHUNK
        out_rows = NC
        den_tiles = NC * NS
    nslice = NPAD // NS
    ncopies = nslice // CHUNK

    def body(h_hbm, src_hbm, tgt_hbm, raw_hbm, tmax_hbm, out_hbm, den_hbm,
             acc, srcqA, srcqB, tgtqA, tgtqB, rawqA, rawqB,
             rows0, rows1, outb0, outb1, denb, tmr,
             gsem0, gsem1, ssem0, ssem1, isemA, isemB):
        setA = (srcqA, tgtqA, rawqA, isemA)
        setB = (srcqB, tgtqB, rawqB, isemB)
        rows = (rows0, rows1)
        outb = (outb0, outb1)
        gsem = (gsem0, gsem1)
        ssem = (ssem0, ssem1)
        cid = lax.axis_index("c")
        sid = lax.axis_index("s")
        head_base = cid * hpc if hpc < n_heads else 0
        if hpc < n_heads:
            edge_base = sid * (EPAD // NS)
            den_slot = sid
        else:
            edge_base = (cid * NS + sid) * (EPAD // (NS * NC))
            den_slot = cid * NS + sid
        row_base = edge_base // CHUNK
        nch = chunks_per_tile

        pltpu.sync_copy(tmax_hbm, tmr)
        mv = jnp.full((16,), _NEG, _f32)
        for r in range(NC * NS):
            mv = jnp.maximum(mv, tmr[r, :])

        def zrow(ci, carry):
            z = jnp.zeros((16,), _f32)
            for j in range(HID // 16):
                outb0[ci, pl.ds(j * 16, 16)] = z
            return carry

        def zden(ci, carry):
            denb[pl.ds(ci * 16, 16)] = jnp.zeros((16,), _f32)
            return carry

        nquads = nch // 4

        for k in range(hpc):
            head = head_base + k
            lax.fori_loop(0, CHUNK, zrow, 0)
            for i in range(ACCN // CHUNK // NS + 1):
                zc = jnp.minimum(sid + NS * i, ACCN // CHUNK - 1)
                zofs = pl.multiple_of(zc * CHUNK, CHUNK)
                pltpu.sync_copy(outb0, acc.at[pl.ds(zofs, CHUNK)])
            lax.fori_loop(0, ACCN // 16, zden, 0)
            plsc.subcore_barrier()

            msc = jnp.max(jnp.where(_iota16() == head, mv, _NEG))

            def qrow(Qn):
                rb = jnp.minimum(row_base + Qn * 4, row_base + nch - 4)
                return pl.multiple_of(rb, 4)

            def idxq_start(Qn, s):
                rb = qrow(Qn)
                pltpu.async_copy(src_hbm.at[pl.ds(rb, 4)], s[0], s[3])
                pltpu.async_copy(tgt_hbm.at[pl.ds(rb, 4)], s[1], s[3])
                pltpu.async_copy(raw_hbm.at[head, pl.ds(rb, 4)], s[2], s[3])

            def idxq_wait(Qn, s):
                rb = qrow(Qn)
                pltpu.make_async_copy(src_hbm.at[pl.ds(rb, 4)],
                                      s[0], s[3]).wait()
                pltpu.make_async_copy(tgt_hbm.at[pl.ds(rb, 4)],
                                      s[1], s[3]).wait()
                pltpu.make_async_copy(raw_hbm.at[head, pl.ds(rb, 4)],
                                      s[2], s[3]).wait()

            def idxq_sync(Qn, s):
                rb = qrow(Qn)
                pltpu.sync_copy(src_hbm.at[pl.ds(rb, 4)], s[0])
                pltpu.sync_copy(tgt_hbm.at[pl.ds(rb, 4)], s[1])
                pltpu.sync_copy(raw_hbm.at[head, pl.ds(rb, 4)], s[2])

            def gs(s, q, p):
                pltpu.async_copy(h_hbm.at[head].at[s[1].at[q]], rows[p],
                                 gsem[p])

            def gw(s, q, p):
                pltpu.make_async_copy(h_hbm.at[head].at[s[1].at[q]], rows[p],
                                      gsem[p]).wait()

            def ss(s, q, p):
                pltpu.async_copy(outb[p], acc.at[s[0].at[q]], ssem[p],
                                 add=True)

            def sw(s, q, p):
                pltpu.make_async_copy(outb[p], acc.at[s[0].at[q]],
                                      ssem[p]).wait()

            def compute(s, q, p):
                def group(g, carry):
                    gofs = pl.multiple_of(g * 16, 16)
                    r16 = s[2][q, pl.ds(gofs, 16)]
                    w16 = jnp.exp(r16 - msc)
                    si16 = s[0][q, pl.ds(gofs, 16)]
                    plsc.addupdate_scatter(denb, [si16], w16)
                    for c2 in range(16):
                        wsc = w16[c2]
                        rr = gofs + c2
                        for j in range(HID // 16):
                            outb[p][rr, pl.ds(j * 16, 16)] = (
                                rows[p][rr, pl.ds(j * 16, 16)] * wsc)
                    return carry
                lax.fori_loop(0, CHUNK // 16, group, 0)

            def sub(cur, nxt, q, Qnext):
                p = q & 1
                gw(cur, q, p)
                if q < 3:
                    gs(cur, q + 1, 1 - p)
                else:
                    idxq_wait(Qnext, nxt)
                    gs(nxt, 0, 1 - p)
                sw(cur, q, p)
                compute(cur, q, p)
                ss(cur, q, p)

            idxq_sync(0, setA)
            gs(setA, 0, 0)
            pltpu.async_copy(outb0, acc.at[setA[0].at[0]], ssem0, add=True)
            pltpu.async_copy(outb0, acc.at[setA[0].at[1]], ssem1, add=True)

            def quadpair(m, carry):
                QB = 2 * m + 1
                sub(setA, setB, 0, QB)
                sub(setA, setB, 1, QB)
                idxq_start(QB, setB)
                sub(setA, setB, 2, QB)
                sub(setA, setB, 3, QB)
                QA2 = 2 * m + 2
                sub(setB, setA, 0, QA2)
                sub(setB, setA, 1, QA2)
                idxq_start(QA2, setA)
                sub(setB, setA, 2, QA2)
                sub(setB, setA, 3, QA2)
                return carry

            lax.fori_loop(0, nquads // 2, quadpair, 0)
            sw(setA, 0, 0)
            sw(setA, 1, 1)
            gw(setA, 0, 0)
            plsc.subcore_barrier()

            out_row = head if hpc < n_heads else cid
            dslice = ACCN // NS
            rr = sid * dslice
            pltpu.sync_copy(acc.at[pl.ds(rr, dslice)],
                            out_hbm.at[out_row, pl.ds(rr, dslice)])
            pltpu.sync_copy(denb, den_hbm.at[head, den_slot, pl.ds(0, ACCN)])
            plsc.subcore_barrier()

    return pl.kernel(
        body,
        out_type=[
            jax.ShapeDtypeStruct((out_rows, NPAD, HID), _f32),
            jax.ShapeDtypeStruct((n_heads, den_tiles, NPAD), _f32),
        ],
        mesh=_MESH,
        compiler_params=_SC_PARAMS,
        scratch_types=[
            pltpu.VMEM_SHARED((ACCN, HID), _f32),
            pltpu.VMEM((4, CHUNK), _i32),
            pltpu.VMEM((4, CHUNK), _i32),
            pltpu.VMEM((4, CHUNK), _i32),
            pltpu.VMEM((4, CHUNK), _i32),
            pltpu.VMEM((4, CHUNK), _f32),
            pltpu.VMEM((4, CHUNK), _f32),
            pltpu.VMEM((CHUNK, HID), _f32),
            pltpu.VMEM((CHUNK, HID), _f32),
            pltpu.VMEM((CHUNK, HID), _f32),
            pltpu.VMEM((CHUNK, HID), _f32),
            pltpu.VMEM((ACCN,), _f32),
            pltpu.VMEM((NC * NS, 16), _f32),
            pltpu.SemaphoreType.DMA,
            pltpu.SemaphoreType.DMA,
            pltpu.SemaphoreType.DMA,
            pltpu.SemaphoreType.DMA,
            pltpu.SemaphoreType.DMA,
            pltpu.SemaphoreType.DMA,
        ],
    )


_score_l1 = _make_score_kernel(N_HEADS, N_HEADS // NC)
_score_l2 = _make_score_kernel(1, 1)
_agg_l1 = _make_agg_kernel(N_HEADS, N_HEADS // NC)
_agg_l2 = _make_agg_kernel(1, 1)


@jax.jit
def _run(X, edge_index, edge_attr, W_heads, a_heads, W_out, a_out):
    Xp = jnp.pad(X, ((0, NPAD - N_NODES), (0, 0)))
    src = jnp.pad(edge_index[0], (0, EPAD - E_EDGES))
    tgt = jnp.pad(edge_index[1], (0, EPAD - E_EDGES))
    eap = jnp.pad(edge_attr, ((0, EPAD - E_EDGES), (0, 0)))

    H1 = _tc_heads_matmul(Xp, W_heads)
    S1 = _tc_node_scores(Xp, W_heads, a_heads)
    A_all = jnp.concatenate(
        [a_heads[:, 2 * HID:, 0], a_out[2 * HID:, 0][None]], axis=0)
    Es_all = _tc_edge_scores(A_all, eap)
    Es1 = Es_all[:N_HEADS]
    Es2 = Es_all[N_HEADS:]

    raw1, tmax1 = _score_l1(S1, src, tgt, Es1)
    src2d = src.reshape(EROWS, CHUNK)
    tgt2d = tgt.reshape(EROWS, CHUNK)
    out_l1, den_l1 = _agg_l1(H1, src2d, tgt2d,
                             raw1.reshape(N_HEADS, EROWS, CHUNK), tmax1)

    H2, S2 = _tc_mid(out_l1, den_l1, W_out, a_out)

    raw2, tmax2 = _score_l2(S2, src, tgt, Es2)
    partials, den_l2 = _agg_l2(H2[None], src2d, tgt2d,
                               raw2.reshape(1, EROWS, CHUNK), tmax2)

    out = _tc_final(partials, den_l2)
    return out[:N_NODES]


def kernel(X, edge_index, edge_attr, W_heads, a_heads, W_out, a_out):
    return _run(X, edge_index, edge_attr, W_heads, a_heads, W_out, a_out)
